# baseline (speedup 1.0000x reference)
import jax
import jax.numpy as jnp
from jax import lax
from jax.experimental import pallas as pl
from jax.experimental.pallas import tpu as pltpu

_INF = float("inf")
_BIG = 2**30


def _knn_body(frames_ref, center_ref, md_ref, mi_ref, dist_ref, colsum_ref,
              rot_ref, rmin_ref):
    t = pl.program_id(0)

    for c in range(3):
        for x in range(22):
            rot_ref[c, x] = frames_ref[0, c, :, x:x + 75]

    for y in range(22):
        prefix = None
        ps = []
        for x in range(22):
            s = None
            for c in range(3):
                diff = rot_ref[c, x, y:y + 75, :] - center_ref[c, y, x]
                sq = diff * diff
                s = sq if s is None else s + sq
            s = s / 3.0
            prefix = s if prefix is None else prefix + s
            ps.append(prefix)
        for qx in range(8):
            cs = ps[qx + 14] if qx == 0 else ps[qx + 14] - ps[qx - 1]
            colsum_ref[y, qx] = cs
    for qx in range(8):
        run = None
        py = []
        for y in range(22):
            v = colsum_ref[y, qx]
            run = v if run is None else run + v
            py.append(run)
        for qy in range(8):
            dist_ref[t, qy, qx] = py[qy + 14] if qy == 0 else py[qy + 14] - py[qy - 1]

    @pl.when(t == 6)
    def _():
        l75 = lax.broadcasted_iota(jnp.int32, (1, 75), 1)

        for qy in range(8):
            for qx in range(8):
                row = dist_ref[3, qy, qx, 37:38, :]
                dist_ref[3, qy, qx, 37:38, :] = jnp.where(l75 == 37, _INF, row)

        for t_ in range(7):
            rmin_ref[t_] = jnp.min(dist_ref[t_], axis=-1)

        tv_iota = (lax.broadcasted_iota(jnp.int32, (7, 8, 8, 75), 0) * 75
                   + lax.broadcasted_iota(jnp.int32, (7, 8, 8, 75), 3))

        def round_body(j, carry):
            rv = rmin_ref[...]
            m = rv.min(axis=0).min(axis=-1)
            itv = jnp.where(rv == m[None, :, :, None], tv_iota, _BIG)
            itv = itv.min(axis=0).min(axis=-1)
            md_ref[j] = m
            for qy in range(8):
                regs = []
                for qx in range(8):
                    it_s = itv[qy, qx]
                    t_s = it_s // 75
                    vs_s = it_s - t_s * 75
                    mq = m[qy, qx]
                    row = dist_ref[t_s, qy, qx, pl.ds(vs_s, 1), :]
                    hs_s = jnp.min(jnp.where(row == mq, l75, _BIG))
                    new_row = jnp.where(l75 == hs_s, _INF, row)
                    m2 = jnp.min(new_row)
                    rrow = rmin_ref[t_s, qy, pl.ds(qx, 1), :]
                    new_rrow = jnp.where(l75 == vs_s, m2, rrow)
                    base = 3 * 6724 + (37 + qy) * 82 + (37 + qx)
                    mi_s = base + (t_s - 3) * 6724 + (vs_s - 37) * 82 + (hs_s - 37)
                    regs.append((t_s, vs_s, new_row, new_rrow, mi_s))
                for qx in range(8):
                    t_s, vs_s, new_row, new_rrow, mi_s = regs[qx]
                    dist_ref[t_s, qy, qx, pl.ds(vs_s, 1), :] = new_row
                    rmin_ref[t_s, qy, pl.ds(qx, 1), :] = new_rrow
                    mi_ref[j, qy:qy + 1, qx:qx + 1] = mi_s[None, None]
            return carry

        lax.fori_loop(0, 14, round_body, 0)


def kernel(seq_pad):
    frames = jnp.transpose(seq_pad[0], (1, 0, 2, 3))
    center = frames[3, :, 37:59, 37:59]
    md, mi = pl.pallas_call(
        _knn_body,
        grid=(7,),
        in_specs=[
            pl.BlockSpec((1, 3, 96, 96), lambda t: (t, 0, 0, 0)),
            pl.BlockSpec((3, 22, 22), lambda t: (0, 0, 0)),
        ],
        out_specs=[
            pl.BlockSpec((14, 8, 8), lambda t: (0, 0, 0)),
            pl.BlockSpec((14, 8, 8), lambda t: (0, 0, 0)),
        ],
        out_shape=[
            jax.ShapeDtypeStruct((14, 8, 8), jnp.float32),
            jax.ShapeDtypeStruct((14, 8, 8), jnp.int32),
        ],
        scratch_shapes=[
            pltpu.VMEM((7, 8, 8, 75, 75), jnp.float32),
            pltpu.VMEM((22, 8, 75, 75), jnp.float32),
            pltpu.VMEM((3, 22, 96, 75), jnp.float32),
            pltpu.VMEM((7, 8, 8, 75), jnp.float32),
        ],
        compiler_params=pltpu.CompilerParams(
            vmem_limit_bytes=100 * 1024 * 1024,
        ),
    )(frames, center)
    md = jnp.transpose(md, (1, 2, 0)).reshape(1, 1, 1, 8, 8, 14)
    mi = jnp.transpose(mi, (1, 2, 0)).reshape(1, 1, 1, 8, 8, 14)
    return md, mi


# speedup vs baseline: 3.4682x; 3.4682x over previous
import jax
import jax.numpy as jnp
from jax import lax
from jax.experimental import pallas as pl
from jax.experimental.pallas import tpu as pltpu

_INF = float("inf")
_BIG = 2**30


def _knn_body(frames_ref, center_ref, md_ref, mi_ref, dist_ref, colsum_ref,
              rot_ref, rmin_ref, itv_ref, hs_ref):
    t = pl.program_id(0)

    for c in range(3):
        for x in range(22):
            rot_ref[c, x] = frames_ref[0, c, :, x:x + 75]

    for y in range(22):
        prefix = None
        ps = []
        for x in range(22):
            s = None
            for c in range(3):
                diff = rot_ref[c, x, y:y + 75, :] - center_ref[c, y, x]
                sq = diff * diff
                s = sq if s is None else s + sq
            s = s / 3.0
            prefix = s if prefix is None else prefix + s
            ps.append(prefix)
        for qx in range(8):
            cs = ps[qx + 14] if qx == 0 else ps[qx + 14] - ps[qx - 1]
            colsum_ref[y, qx] = cs
    for qx in range(8):
        run = None
        py = []
        for y in range(22):
            v = colsum_ref[y, qx]
            run = v if run is None else run + v
            py.append(run)
        for qy in range(8):
            dist_ref[t, qy, qx] = py[qy + 14] if qy == 0 else py[qy + 14] - py[qy - 1]

    @pl.when(t == 6)
    def _():
        l75 = lax.broadcasted_iota(jnp.int32, (1, 75), 1)

        for qy in range(8):
            for qx in range(8):
                row = dist_ref[3, qy, qx, 37:38, :]
                dist_ref[3, qy, qx, 37:38, :] = jnp.where(l75 == 37, _INF, row)

        for t_ in range(7):
            rmin_ref[t_] = jnp.min(dist_ref[t_], axis=-1)

        tv_iota = (lax.broadcasted_iota(jnp.int32, (7, 8, 8, 75), 0) * 75
                   + lax.broadcasted_iota(jnp.int32, (7, 8, 8, 75), 3))

        def round_body(j, carry):
            rv = rmin_ref[...]
            m = rv.min(axis=0).min(axis=-1)
            itv = jnp.where(rv == m[None, :, :, None], tv_iota, _BIG)
            itv = itv.min(axis=0).min(axis=-1)
            md_ref[j] = m
            itv_ref[j] = itv
            for qy in range(8):
                regs = []
                for qx in range(8):
                    it_s = itv[qy, qx]
                    t_s = it_s // 75
                    vs_s = it_s - t_s * 75
                    row = dist_ref[t_s, qy, qx, pl.ds(vs_s, 1), :]
                    eq = row == jnp.min(row, axis=-1, keepdims=True)
                    hs_v = jnp.min(jnp.where(eq, l75, _BIG), axis=-1,
                                   keepdims=True)
                    new_row = jnp.where(l75 == hs_v, _INF, row)
                    m2 = jnp.min(new_row, axis=-1, keepdims=True)
                    rrow = rmin_ref[t_s, qy, pl.ds(qx, 1), :]
                    new_rrow = jnp.where(l75 == vs_s, m2, rrow)
                    regs.append((t_s, vs_s, new_row, new_rrow, hs_v))
                for qx in range(8):
                    t_s, vs_s, new_row, new_rrow, hs_v = regs[qx]
                    dist_ref[t_s, qy, qx, pl.ds(vs_s, 1), :] = new_row
                    rmin_ref[t_s, qy, pl.ds(qx, 1), :] = new_rrow
                    hs_ref[j, qy:qy + 1, qx:qx + 1] = hs_v
            return carry

        lax.fori_loop(0, 14, round_body, 0)

        qyv = lax.broadcasted_iota(jnp.int32, (8, 8), 0)
        qxv = lax.broadcasted_iota(jnp.int32, (8, 8), 1)
        base = 3 * 6724 + (37 + qyv) * 82 + (37 + qxv)
        tvw = itv_ref[...]
        tw = tvw // 75
        vw = tvw - tw * 75
        hw = hs_ref[...]
        mi_ref[...] = (base[None] + (tw - 3) * 6724 + (vw - 37) * 82
                       + (hw - 37))


def kernel(seq_pad):
    frames = jnp.transpose(seq_pad[0], (1, 0, 2, 3))
    center = frames[3, :, 37:59, 37:59]
    md, mi = pl.pallas_call(
        _knn_body,
        grid=(7,),
        in_specs=[
            pl.BlockSpec((1, 3, 96, 96), lambda t: (t, 0, 0, 0)),
            pl.BlockSpec((3, 22, 22), lambda t: (0, 0, 0)),
        ],
        out_specs=[
            pl.BlockSpec((14, 8, 8), lambda t: (0, 0, 0)),
            pl.BlockSpec((14, 8, 8), lambda t: (0, 0, 0)),
        ],
        out_shape=[
            jax.ShapeDtypeStruct((14, 8, 8), jnp.float32),
            jax.ShapeDtypeStruct((14, 8, 8), jnp.int32),
        ],
        scratch_shapes=[
            pltpu.VMEM((7, 8, 8, 75, 75), jnp.float32),
            pltpu.VMEM((22, 8, 75, 75), jnp.float32),
            pltpu.VMEM((3, 22, 96, 75), jnp.float32),
            pltpu.VMEM((7, 8, 8, 75), jnp.float32),
            pltpu.VMEM((14, 8, 8), jnp.int32),
            pltpu.VMEM((14, 8, 8), jnp.int32),
        ],
        compiler_params=pltpu.CompilerParams(
            vmem_limit_bytes=100 * 1024 * 1024,
        ),
    )(frames, center)
    md = jnp.transpose(md, (1, 2, 0)).reshape(1, 1, 1, 8, 8, 14)
    mi = jnp.transpose(mi, (1, 2, 0)).reshape(1, 1, 1, 8, 8, 14)
    return md, mi
